# Initial kernel scaffold; baseline (speedup 1.0000x reference)
import jax
import jax.numpy as jnp
from jax.experimental import pallas as pl


def kernel(X, embedding_weight):
    raise NotImplementedError("write your pallas kernel here")


# speedup vs baseline: 2.9681x; 2.9681x over previous
import functools

import jax
import jax.numpy as jnp
from jax import lax
from jax.experimental import pallas as pl
from jax.experimental.pallas import tpu as pltpu
from jax.experimental.pallas import tpu_sc as plsc

_NC = 2
_NS = 16
_NW = _NC * _NS
_CHUNK = 128


def _gather_body(ch, embed, idx_hbm, table_hbm, out_hbm, idx_v, rows_v, sem):
    wid = lax.axis_index("s") * _NC + lax.axis_index("c")
    pltpu.sync_copy(idx_hbm.at[wid], idx_v)
    base = wid * (ch * _CHUNK)

    def chunk_fn(j, carry):
        pltpu.async_copy(table_hbm.at[idx_v.at[j]], rows_v, sem).wait()
        pltpu.sync_copy(rows_v, out_hbm.at[pl.ds(base + j * _CHUNK, _CHUNK)])
        return carry

    lax.fori_loop(0, ch, chunk_fn, 0)


@functools.partial(jax.jit, static_argnums=(2, 3))
def _sc_gather(idx, table, ch, embed):
    n = _NW * ch * _CHUNK
    mesh = plsc.VectorSubcoreMesh(core_axis_name="c", subcore_axis_name="s")
    fn = pl.kernel(
        functools.partial(_gather_body, ch, embed),
        mesh=mesh,
        out_type=jax.ShapeDtypeStruct((n, embed), jnp.float32),
        scratch_types=[
            pltpu.VMEM((ch, _CHUNK), jnp.int32),
            pltpu.VMEM((_CHUNK, embed), jnp.float32),
            pltpu.SemaphoreType.DMA,
        ],
    )
    return fn(idx, table)


def kernel(X, embedding_weight):
    b, s = X.shape
    vocab, embed = embedding_weight.shape
    n = b * s
    ch = n // (_NW * _CHUNK)
    idx = X.reshape(_NW, ch, _CHUNK).astype(jnp.int32)
    out = _sc_gather(idx, embedding_weight, ch, embed)
    return out.reshape(b, s, embed)


# speedup vs baseline: 3.3438x; 1.1266x over previous
import functools

import jax
import jax.numpy as jnp
from jax import lax
from jax.experimental import pallas as pl
from jax.experimental.pallas import tpu as pltpu
from jax.experimental.pallas import tpu_sc as plsc

_NC = 2
_NS = 16
_NW = _NC * _NS
_CHUNK = 128
_NB = 5
_A = 3


def _gather_body(ch, embed, idx_hbm, table_hbm, out_hbm, idx_v, rows_v,
                 gsem, wsem):
    wid = lax.axis_index("s") * _NC + lax.axis_index("c")
    pltpu.sync_copy(idx_hbm.at[wid], idx_v)
    base = wid * (ch * _CHUNK)

    def gather(c, b):
        return pltpu.make_async_copy(
            table_hbm.at[idx_v.at[c]], rows_v.at[b], gsem.at[b])

    def write(c, b):
        return pltpu.make_async_copy(
            rows_v.at[b], out_hbm.at[pl.ds(base + c * _CHUNK, _CHUNK)],
            wsem.at[b])

    def step(c, b, do_wait_w, do_gather):
        gather(c, b).wait()
        write(c, b).start()
        f = c + _A
        bf = (b + _A) % _NB
        if do_wait_w:
            write(f - _NB, bf).wait()
        if do_gather:
            gather(f, bf).start()

    for r in range(_A):
        gather(r, r).start()

    for r in range(_NB):
        step(r, r, do_wait_w=(r + _A >= _NB), do_gather=True)

    def outer(j, carry):
        c0 = j * _NB
        for r in range(_NB):
            step(c0 + r, r, do_wait_w=True, do_gather=True)
        return carry

    lax.fori_loop(1, ch // _NB - 1, outer, 0)

    for r in range(_NB):
        c = ch - _NB + r
        step(c, r, do_wait_w=(r + _A < _NB), do_gather=(r + _A < _NB))

    for b in range(_NB):
        write(ch - _NB + b, b).wait()


@functools.partial(jax.jit, static_argnums=(2, 3))
def _sc_gather(idx, table, ch, embed):
    n = _NW * ch * _CHUNK
    mesh = plsc.VectorSubcoreMesh(core_axis_name="c", subcore_axis_name="s")
    fn = pl.kernel(
        functools.partial(_gather_body, ch, embed),
        mesh=mesh,
        out_type=jax.ShapeDtypeStruct((n, embed), jnp.float32),
        scratch_types=[
            pltpu.VMEM((ch, _CHUNK), jnp.int32),
            pltpu.VMEM((_NB, _CHUNK, embed), jnp.float32),
            pltpu.SemaphoreType.DMA((_NB,)),
            pltpu.SemaphoreType.DMA((_NB,)),
        ],
    )
    return fn(idx, table)


def kernel(X, embedding_weight):
    b, s = X.shape
    vocab, embed = embedding_weight.shape
    n = b * s
    ch = n // (_NW * _CHUNK)
    idx = X.reshape(_NW, ch, _CHUNK).astype(jnp.int32)
    out = _sc_gather(idx, embedding_weight, ch, embed)
    return out.reshape(b, s, embed)


# speedup vs baseline: 5.9577x; 1.7817x over previous
import functools

import jax
import jax.numpy as jnp
from jax import lax
from jax.experimental import pallas as pl
from jax.experimental.pallas import tpu as pltpu
from jax.experimental.pallas import tpu_sc as plsc

_NC = 2
_NS = 16
_NW = _NC * _NS
_NB = 8
_A = 6


def _gather_body(ch, seq, embed, idx_hbm, table_hbm, out_hbm, idx_v, rows_v,
                 gsem, wsem):
    wid = lax.axis_index("s") * _NC + lax.axis_index("c")
    pltpu.sync_copy(idx_hbm.at[wid], idx_v)
    base = wid * ch

    def gather(c, b):
        return pltpu.make_async_copy(
            table_hbm.at[idx_v.at[c]], rows_v.at[b], gsem.at[b])

    def write(c, b):
        return pltpu.make_async_copy(
            rows_v.at[b], out_hbm.at[base + c], wsem.at[b])

    def step(c, b, do_wait_w, do_gather):
        gather(c, b).wait()
        write(c, b).start()
        f = c + _A
        bf = (b + _A) % _NB
        if do_wait_w:
            write(f - _NB, bf).wait()
        if do_gather:
            gather(f, bf).start()

    for r in range(_A):
        gather(r, r % _NB).start()

    for r in range(_NB):
        step(r, r, do_wait_w=(r + _A >= _NB), do_gather=True)

    def outer(j, carry):
        c0 = j * _NB
        for r in range(_NB):
            step(c0 + r, r, do_wait_w=True, do_gather=True)
        return carry

    lax.fori_loop(1, ch // _NB - 1, outer, 0)

    for r in range(_NB):
        step(ch - _NB + r, r, do_wait_w=(r + _A < _NB),
             do_gather=(r + _A < _NB))

    for b in range(_NB):
        write(ch - _NB + b, b).wait()


@functools.partial(jax.jit, static_argnums=(2, 3, 4))
def _sc_gather(idx, table, ch, seq, embed):
    mesh = plsc.VectorSubcoreMesh(core_axis_name="c", subcore_axis_name="s")
    fn = pl.kernel(
        functools.partial(_gather_body, ch, seq, embed),
        mesh=mesh,
        out_type=jax.ShapeDtypeStruct((_NW * ch, seq, embed), jnp.float32),
        scratch_types=[
            pltpu.VMEM((ch, seq), jnp.int32),
            pltpu.VMEM((_NB, seq, embed), jnp.float32),
            pltpu.SemaphoreType.DMA((_NB,)),
            pltpu.SemaphoreType.DMA((_NB,)),
        ],
    )
    return fn(idx, table)


def kernel(X, embedding_weight):
    b, s = X.shape
    vocab, embed = embedding_weight.shape
    ch = b // _NW
    idx = X.reshape(_NW, ch, s).astype(jnp.int32)
    return _sc_gather(idx, embedding_weight, ch, s, embed)


# speedup vs baseline: 5.9663x; 1.0014x over previous
import functools

import jax
import jax.numpy as jnp
from jax import lax
from jax.experimental import pallas as pl
from jax.experimental.pallas import tpu as pltpu
from jax.experimental.pallas import tpu_sc as plsc

_NC = 2
_NS = 16
_NW = _NC * _NS
_NB = 8
_A = 6


def _gather_body(ch, seq, embed, idx_hbm, table_hbm, out_hbm, idx_v, rows_v,
                 gsem, wsem):
    wid = lax.axis_index("s") * _NC + lax.axis_index("c")
    pltpu.sync_copy(idx_hbm.at[wid], idx_v)
    base = wid * ch

    def gather(c, b):
        return pltpu.make_async_copy(
            table_hbm.at[idx_v.at[c]], rows_v.at[b], gsem.at[b])

    def write(c, b):
        return pltpu.make_async_copy(
            rows_v.at[b], out_hbm.at[base + c], wsem.at[b])

    def step(c, b, do_wait_w, do_gather):
        gather(c, b).wait()
        write(c, b).start()
        f = c + _A
        bf = (b + _A) % _NB
        if do_wait_w:
            write(f - _NB, bf).wait()
        if do_gather:
            gather(f, bf).start()

    for r in range(_A):
        gather(r, r % _NB).start()

    for r in range(_NB):
        step(r, r, do_wait_w=(r + _A >= _NB), do_gather=True)

    def outer(j, carry):
        c0 = j * _NB
        for r in range(_NB):
            step(c0 + r, r, do_wait_w=True, do_gather=True)
        return carry

    lax.fori_loop(1, ch // _NB - 1, outer, 0)

    for r in range(_NB):
        step(ch - _NB + r, r, do_wait_w=(r + _A < _NB),
             do_gather=(r + _A < _NB))

    for b in range(_NB):
        write(ch - _NB + b, b).wait()


@functools.partial(jax.jit, static_argnums=(2, 3, 4))
def _sc_gather(idx, table, ch, seq, embed):
    mesh = plsc.VectorSubcoreMesh(core_axis_name="c", subcore_axis_name="s")
    fn = pl.kernel(
        functools.partial(_gather_body, ch, seq, embed),
        mesh=mesh,
        out_type=jax.ShapeDtypeStruct((_NW * ch, seq, embed), jnp.float32),
        scratch_types=[
            pltpu.VMEM((ch, seq), jnp.int32),
            pltpu.VMEM((_NB, seq, embed), jnp.float32),
            pltpu.SemaphoreType.DMA((_NB,)),
            pltpu.SemaphoreType.DMA((_NB,)),
        ],
        compiler_params=pltpu.CompilerParams(use_tc_tiling_on_sc=True),
    )
    return fn(idx, table)


def kernel(X, embedding_weight):
    b, s = X.shape
    vocab, embed = embedding_weight.shape
    ch = b // _NW
    idx = X.reshape(_NW, ch, s).astype(jnp.int32)
    return _sc_gather(idx, embedding_weight, ch, s, embed)
